# Initial kernel scaffold; baseline (speedup 1.0000x reference)
import jax
import jax.numpy as jnp
from jax.experimental import pallas as pl


def kernel(freq, edge_index, edge_weight, cheb_W0, cheb_b0, cheb_W1, cheb_b1, cheb_W2, cheb_b2, pool_w, mlp_W1, mlp_b1, mlp_W2, mlp_b2):
    raise NotImplementedError("write your pallas kernel here")


# speedup vs baseline: 54.9079x; 54.9079x over previous
import functools
import math

import jax
import jax.numpy as jnp
from jax import lax
from jax.experimental import pallas as pl
from jax.experimental.pallas import tpu as pltpu
from jax.experimental.pallas import tpu_sc as plsc

NP = 1280
NTILE = 16
NCORE = 2
CHUNK = 128
NCH = 20
ZB = 12800
SLICE = NP * NP // NTILE


def _sc_body(src_hbm, dst_hbm, val_hbm, out_hbm,
             src_v, dst_v, val_v, idx_v, zbuf, acc_shared):
    cid = lax.axis_index("c")
    sid = lax.axis_index("s")
    nb = src_hbm.shape[0]
    gpc = nb // NCORE

    def _z(i, _):
        zbuf[pl.ds(i * 16, 16)] = jnp.zeros((16,), jnp.float32)
        return 0
    lax.fori_loop(0, ZB // 16, _z, 0)

    def _graph(gi, _):
        g = cid * gpc + gi
        pltpu.sync_copy(src_hbm.at[g, sid], src_v)
        pltpu.sync_copy(dst_hbm.at[g, sid], dst_v)
        pltpu.sync_copy(val_hbm.at[g, sid], val_v)
        base = sid * SLICE
        for j in range(SLICE // ZB):
            pltpu.sync_copy(zbuf, acc_shared.at[pl.ds(base + j * ZB, ZB)])
        for j in range(NCH):
            def _idx(k, _, j=j):
                sl = pl.ds(k * 16, 16)
                idx_v[j, sl] = dst_v[j, sl] * NP + src_v[j, sl]
                return 0
            lax.fori_loop(0, CHUNK // 16, _idx, 0)
        plsc.subcore_barrier()
        for j in range(NCH):
            pltpu.sync_copy(val_v.at[j], acc_shared.at[idx_v.at[j]], add=True)
        plsc.subcore_barrier()
        pltpu.sync_copy(acc_shared.at[pl.ds(base, SLICE)],
                        out_hbm.at[g, pl.ds(base, SLICE)])
        return 0

    lax.fori_loop(0, gpc, _graph, 0)


def _sc_build(srcp, dstp, valp):
    nb = srcp.shape[0]
    mesh = plsc.VectorSubcoreMesh(core_axis_name="c", subcore_axis_name="s")
    return pl.kernel(
        _sc_body,
        out_type=jax.ShapeDtypeStruct((nb, NP * NP), jnp.float32),
        mesh=mesh,
        scratch_types=[
            pltpu.VMEM((NCH, CHUNK), jnp.int32),
            pltpu.VMEM((NCH, CHUNK), jnp.int32),
            pltpu.VMEM((NCH, CHUNK), jnp.float32),
            pltpu.VMEM((NCH, CHUNK), jnp.int32),
            pltpu.VMEM((ZB,), jnp.float32),
            pltpu.VMEM_SHARED((NP * NP,), jnp.float32),
        ],
    )(srcp, dstp, valp)


def _count_ge(key, trial):
    return jnp.sum(jnp.where(key >= trial, jnp.int32(1), jnp.int32(0)))


def _tc_body(nvalid, kth, wm_ref, x_ref, w0_ref, b0_ref, w1_ref, b1_ref,
             w2_ref, b2_ref, pw_ref, m1_ref, mb1_ref, m2_ref, mb2_ref,
             out_ref):
    f32 = jnp.float32
    wm = wm_ref[0]
    ones = jnp.ones((NP, 1), f32)
    deg_row = jnp.sum(wm, axis=0, keepdims=True)
    deg_col = lax.dot_general(wm, ones, (((0,), (0,)), ((), ())))
    dis_row = jnp.where(deg_row > 0,
                        lax.rsqrt(jnp.where(deg_row > 0, deg_row, 1.0)), 0.0)
    dis_col = jnp.where(deg_col > 0,
                        lax.rsqrt(jnp.where(deg_col > 0, deg_col, 1.0)), 0.0)
    a = -(wm * dis_row) * dis_col

    h = x_ref[0]
    for w_ref, b_ref, last in ((w0_ref, b0_ref, False),
                               (w1_ref, b1_ref, False),
                               (w2_ref, b2_ref, True)):
        tx0 = h
        tx1 = jnp.dot(a, tx0, preferred_element_type=f32)
        tx2 = 2.0 * jnp.dot(a, tx1, preferred_element_type=f32) - tx0
        out = (jnp.dot(tx0, w_ref[0], preferred_element_type=f32)
               + jnp.dot(tx1, w_ref[1], preferred_element_type=f32)
               + jnp.dot(tx2, w_ref[2], preferred_element_type=f32)
               + b_ref[...])
        h = out if last else jnp.maximum(out, 0.0)

    pw = pw_ref[...]
    invn = lax.rsqrt(jnp.sum(pw * pw))
    s = jnp.tanh(lax.dot_general(h, pw, (((1,), (1,)), ((), ()))) * invn)
    bits = lax.bitcast_convert_type(s, jnp.int32)
    key = jnp.where(bits < 0, jnp.bitwise_xor(bits, jnp.int32(0x7FFFFFFF)),
                    bits)
    row = lax.broadcasted_iota(jnp.int32, (NP, 1), 0)
    int_min = jnp.int32(-2147483647 - 1)
    key = jnp.where(row < nvalid, key, int_min)

    cpos = _count_ge(key, jnp.int32(0))
    ans0 = jnp.where(cpos >= kth, jnp.int32(0), int_min)

    def _bit(i, ans):
        trial = ans + lax.shift_left(jnp.int32(1), jnp.int32(30) - i)
        return jnp.where(_count_ge(key, trial) >= kth, trial, ans)
    thresh = lax.fori_loop(0, 31, _bit, ans0)

    cgt = _count_ge(key, thresh + 1)
    need = jnp.int32(kth) - cgt
    eq = key == thresh

    def _f(j):
        return jnp.sum(jnp.where(eq & (row <= j), jnp.int32(1), jnp.int32(0)))

    def _jbit(i, j):
        cand = j + lax.shift_left(jnp.int32(1), jnp.int32(10) - i)
        return jnp.where(_f(cand - 1) < need, cand, j)
    jsel = lax.fori_loop(0, 11, _jbit, jnp.int32(0))

    keep = (key > thresh) | (eq & (row <= jsel))
    w = jnp.where(keep, s, 0.0) * (1.0 / kth)
    g_row = lax.dot_general(w, h, (((0,), (0,)), ((), ())))

    z = jnp.maximum(jnp.dot(g_row, m1_ref[...], preferred_element_type=f32)
                    + mb1_ref[...], 0.0)
    o = jnp.dot(z, m2_ref[...], preferred_element_type=f32) + mb2_ref[...]
    out_ref[0] = jnp.broadcast_to(o, (8, 128))


def _tc_call(wm, xp, w0, b0, w1, b1, w2, b2, pw, m1, mb1, m2p, mb2p,
             nvalid, kth):
    nb = wm.shape[0]
    d_in = xp.shape[2]
    full = lambda shp: pl.BlockSpec(shp, lambda g: (0,) * len(shp))
    grid_spec = pl.GridSpec(
        grid=(nb,),
        in_specs=[
            pl.BlockSpec((1, NP, NP), lambda g: (g, 0, 0)),
            pl.BlockSpec((1, NP, d_in), lambda g: (g, 0, 0)),
            full(w0.shape), full(b0.shape), full(w1.shape), full(b1.shape),
            full(w2.shape), full(b2.shape), full(pw.shape), full(m1.shape),
            full(mb1.shape), full(m2p.shape), full(mb2p.shape),
        ],
        out_specs=pl.BlockSpec((1, 8, 128), lambda g: (g, 0, 0)),
    )
    body = functools.partial(_tc_body, nvalid, kth)
    return pl.pallas_call(
        body,
        grid_spec=grid_spec,
        out_shape=jax.ShapeDtypeStruct((nb, 8, 128), jnp.float32),
    )(wm, xp, w0, b0, w1, b1, w2, b2, pw, m1, mb1, m2p, mb2p)


def kernel(freq, edge_index, edge_weight, cheb_W0, cheb_b0, cheb_W1, cheb_b1,
           cheb_W2, cheb_b2, pool_w, mlp_W1, mlp_b1, mlp_W2, mlp_b2):
    nb, n, d = freq.shape
    e = edge_index.shape[2]
    ep = NTILE * NCH * CHUNK
    kth = int(math.ceil(0.5 * n))
    ncls = mlp_W2.shape[1]

    src = edge_index[:, 0, :].astype(jnp.int32)
    dst = edge_index[:, 1, :].astype(jnp.int32)
    srcp = jnp.pad(src, ((0, 0), (0, ep - e))).reshape(nb, NTILE, NCH, CHUNK)
    dstp = jnp.pad(dst, ((0, 0), (0, ep - e))).reshape(nb, NTILE, NCH, CHUNK)
    valp = jnp.pad(edge_weight, ((0, 0), (0, ep - e))).reshape(
        nb, NTILE, NCH, CHUNK)

    wm = _sc_build(srcp, dstp, valp).reshape(nb, NP, NP)
    xp = jnp.pad(freq, ((0, 0), (0, NP - n), (0, 0)))

    m2p = jnp.pad(mlp_W2, ((0, 0), (0, 128 - ncls)))
    mb2p = jnp.pad(mlp_b2, ((0, 128 - ncls),)).reshape(1, 128)
    out = _tc_call(wm, xp,
                   cheb_W0, cheb_b0.reshape(1, -1),
                   cheb_W1, cheb_b1.reshape(1, -1),
                   cheb_W2, cheb_b2.reshape(1, -1),
                   pool_w.reshape(1, -1),
                   mlp_W1, mlp_b1.reshape(1, -1), m2p, mb2p,
                   n, kth)
    return out[:, 0, :ncls]


# speedup vs baseline: 64.4210x; 1.1733x over previous
import functools
import math

import jax
import jax.numpy as jnp
from jax import lax
from jax.experimental import pallas as pl
from jax.experimental.pallas import tpu as pltpu
from jax.experimental.pallas import tpu_sc as plsc

NP = 1280
NTILE = 16
NCORE = 2
CHUNK = 128
NCH = 20
ZB = 12800
SLICE = NP * NP // NTILE


def _sc_body(src_hbm, dst_hbm, val_hbm, out_hbm,
             src_v, dst_v, val_v, idx_v, zbuf, acc_shared, sem):
    cid = lax.axis_index("c")
    sid = lax.axis_index("s")
    nb = src_hbm.shape[0]
    gpc = nb // NCORE

    def _z(i, _):
        zbuf[pl.ds(i * 16, 16)] = jnp.zeros((16,), jnp.float32)
        return 0
    lax.fori_loop(0, ZB // 16, _z, 0)

    def _graph(gi, _):
        g = cid * gpc + gi
        pltpu.sync_copy(src_hbm.at[g, sid], src_v)
        pltpu.sync_copy(dst_hbm.at[g, sid], dst_v)
        pltpu.sync_copy(val_hbm.at[g, sid], val_v)
        base = sid * SLICE
        for j in range(SLICE // ZB):
            pltpu.sync_copy(zbuf, acc_shared.at[pl.ds(base + j * ZB, ZB)])
        for j in range(NCH):
            def _idx(k, _, j=j):
                sl = pl.ds(k * 16, 16)
                idx_v[j, sl] = dst_v[j, sl] * NP + src_v[j, sl]
                return 0
            lax.fori_loop(0, CHUNK // 16, _idx, 0)
        plsc.subcore_barrier()
        for j in range(NCH):
            pltpu.sync_copy(val_v.at[j], acc_shared.at[idx_v.at[j]], add=True)
        plsc.subcore_barrier()
        rbase = sid * (NP // NTILE)

        def _fire(r, _):
            pltpu.async_copy(acc_shared.at[pl.ds((rbase + r) * NP, NP)],
                             out_hbm.at[g, rbase + r], sem)
            return 0
        lax.fori_loop(0, NP // NTILE, _fire, 0)

        def _drain(r, _):
            pltpu.make_async_copy(
                acc_shared.at[pl.ds((rbase + r) * NP, NP)],
                out_hbm.at[g, rbase + r], sem).wait()
            return 0
        lax.fori_loop(0, NP // NTILE, _drain, 0)
        return 0

    lax.fori_loop(0, gpc, _graph, 0)


def _sc_build(srcp, dstp, valp):
    nb = srcp.shape[0]
    mesh = plsc.VectorSubcoreMesh(core_axis_name="c", subcore_axis_name="s")
    return pl.kernel(
        _sc_body,
        out_type=jax.ShapeDtypeStruct((nb, NP, NP), jnp.float32),
        mesh=mesh,
        scratch_types=[
            pltpu.VMEM((NCH, CHUNK), jnp.int32),
            pltpu.VMEM((NCH, CHUNK), jnp.int32),
            pltpu.VMEM((NCH, CHUNK), jnp.float32),
            pltpu.VMEM((NCH, CHUNK), jnp.int32),
            pltpu.VMEM((ZB,), jnp.float32),
            pltpu.VMEM_SHARED((NP * NP,), jnp.float32),
            pltpu.SemaphoreType.DMA,
        ],
    )(srcp, dstp, valp)


def _count_ge(key, trial):
    return jnp.sum(jnp.where(key >= trial, jnp.int32(1), jnp.int32(0)))


def _tc_body(nvalid, kth, wm_ref, x_ref, w0_ref, b0_ref, w1_ref, b1_ref,
             w2_ref, b2_ref, pw_ref, m1_ref, mb1_ref, m2_ref, mb2_ref,
             out_ref):
    f32 = jnp.float32
    wm = wm_ref[0]
    ones = jnp.ones((NP, 1), f32)
    deg_row = jnp.sum(wm, axis=0, keepdims=True)
    deg_col = lax.dot_general(wm, ones, (((0,), (0,)), ((), ())))
    dis_row = jnp.where(deg_row > 0,
                        lax.rsqrt(jnp.where(deg_row > 0, deg_row, 1.0)), 0.0)
    dis_col = jnp.where(deg_col > 0,
                        lax.rsqrt(jnp.where(deg_col > 0, deg_col, 1.0)), 0.0)
    a = -(wm * dis_row) * dis_col

    h = x_ref[0]
    for w_ref, b_ref, last in ((w0_ref, b0_ref, False),
                               (w1_ref, b1_ref, False),
                               (w2_ref, b2_ref, True)):
        tx0 = h
        tx1 = jnp.dot(a, tx0, preferred_element_type=f32)
        tx2 = 2.0 * jnp.dot(a, tx1, preferred_element_type=f32) - tx0
        out = (jnp.dot(tx0, w_ref[0], preferred_element_type=f32)
               + jnp.dot(tx1, w_ref[1], preferred_element_type=f32)
               + jnp.dot(tx2, w_ref[2], preferred_element_type=f32)
               + b_ref[...])
        h = out if last else jnp.maximum(out, 0.0)

    pw = pw_ref[...]
    invn = lax.rsqrt(jnp.sum(pw * pw))
    s = jnp.tanh(lax.dot_general(h, pw, (((1,), (1,)), ((), ()))) * invn)
    bits = lax.bitcast_convert_type(s, jnp.int32)
    key = jnp.where(bits < 0, jnp.bitwise_xor(bits, jnp.int32(0x7FFFFFFF)),
                    bits)
    row = lax.broadcasted_iota(jnp.int32, (NP, 1), 0)
    int_min = jnp.int32(-2147483647 - 1)
    key = jnp.where(row < nvalid, key, int_min)

    cpos = _count_ge(key, jnp.int32(0))
    ans0 = jnp.where(cpos >= kth, jnp.int32(0), int_min)

    def _bit(i, ans):
        trial = ans + lax.shift_left(jnp.int32(1), jnp.int32(30) - i)
        return jnp.where(_count_ge(key, trial) >= kth, trial, ans)
    thresh = lax.fori_loop(0, 31, _bit, ans0)

    cgt = _count_ge(key, thresh + 1)
    need = jnp.int32(kth) - cgt
    eq = key == thresh

    def _f(j):
        return jnp.sum(jnp.where(eq & (row <= j), jnp.int32(1), jnp.int32(0)))

    def _jbit(i, j):
        cand = j + lax.shift_left(jnp.int32(1), jnp.int32(10) - i)
        return jnp.where(_f(cand - 1) < need, cand, j)
    jsel = lax.fori_loop(0, 11, _jbit, jnp.int32(0))

    keep = (key > thresh) | (eq & (row <= jsel))
    w = jnp.where(keep, s, 0.0) * (1.0 / kth)
    g_row = lax.dot_general(w, h, (((0,), (0,)), ((), ())))

    z = jnp.maximum(jnp.dot(g_row, m1_ref[...], preferred_element_type=f32)
                    + mb1_ref[...], 0.0)
    o = jnp.dot(z, m2_ref[...], preferred_element_type=f32) + mb2_ref[...]
    out_ref[0] = jnp.broadcast_to(o, (8, 128))


def _tc_call(wm, xp, w0, b0, w1, b1, w2, b2, pw, m1, mb1, m2p, mb2p,
             nvalid, kth):
    nb = wm.shape[0]
    d_in = xp.shape[2]
    full = lambda shp: pl.BlockSpec(shp, lambda g: (0,) * len(shp))
    grid_spec = pl.GridSpec(
        grid=(nb,),
        in_specs=[
            pl.BlockSpec((1, NP, NP), lambda g: (g, 0, 0)),
            pl.BlockSpec((1, NP, d_in), lambda g: (g, 0, 0)),
            full(w0.shape), full(b0.shape), full(w1.shape), full(b1.shape),
            full(w2.shape), full(b2.shape), full(pw.shape), full(m1.shape),
            full(mb1.shape), full(m2p.shape), full(mb2p.shape),
        ],
        out_specs=pl.BlockSpec((1, 8, 128), lambda g: (g, 0, 0)),
    )
    body = functools.partial(_tc_body, nvalid, kth)
    return pl.pallas_call(
        body,
        grid_spec=grid_spec,
        out_shape=jax.ShapeDtypeStruct((nb, 8, 128), jnp.float32),
    )(wm, xp, w0, b0, w1, b1, w2, b2, pw, m1, mb1, m2p, mb2p)


def kernel(freq, edge_index, edge_weight, cheb_W0, cheb_b0, cheb_W1, cheb_b1,
           cheb_W2, cheb_b2, pool_w, mlp_W1, mlp_b1, mlp_W2, mlp_b2):
    nb, n, d = freq.shape
    e = edge_index.shape[2]
    ep = NTILE * NCH * CHUNK
    kth = int(math.ceil(0.5 * n))
    ncls = mlp_W2.shape[1]

    src = edge_index[:, 0, :].astype(jnp.int32)
    dst = edge_index[:, 1, :].astype(jnp.int32)
    srcp = jnp.pad(src, ((0, 0), (0, ep - e))).reshape(nb, NTILE, NCH, CHUNK)
    dstp = jnp.pad(dst, ((0, 0), (0, ep - e))).reshape(nb, NTILE, NCH, CHUNK)
    valp = jnp.pad(edge_weight, ((0, 0), (0, ep - e))).reshape(
        nb, NTILE, NCH, CHUNK)

    wm = _sc_build(srcp, dstp, valp)
    xp = jnp.pad(freq, ((0, 0), (0, NP - n), (0, 0)))

    m2p = jnp.pad(mlp_W2, ((0, 0), (0, 128 - ncls)))
    mb2p = jnp.pad(mlp_b2, ((0, 128 - ncls),)).reshape(1, 128)
    out = _tc_call(wm, xp,
                   cheb_W0, cheb_b0.reshape(1, -1),
                   cheb_W1, cheb_b1.reshape(1, -1),
                   cheb_W2, cheb_b2.reshape(1, -1),
                   pool_w.reshape(1, -1),
                   mlp_W1, mlp_b1.reshape(1, -1), m2p, mb2p,
                   n, kth)
    return out[:, 0, :ncls]


# speedup vs baseline: 72.1259x; 1.1196x over previous
import functools
import math

import jax
import jax.numpy as jnp
from jax import lax
from jax.experimental import pallas as pl
from jax.experimental.pallas import tpu as pltpu
from jax.experimental.pallas import tpu_sc as plsc

NP = 1280
NTILE = 16
NCORE = 2
CHUNK = 128
NCH = 20
ZB = 12800
SLICE = NP * NP // NTILE


def _sc_body(src_hbm, dst_hbm, val_hbm, out_hbm,
             src_v, dst_v, val_v, idx_v, zbuf, acc_shared, sem):
    cid = lax.axis_index("c")
    sid = lax.axis_index("s")
    nb = src_hbm.shape[0]
    gpc = nb // NCORE

    def _z(i, _):
        zbuf[pl.ds(i * 16, 16)] = jnp.zeros((16,), jnp.float32)
        return 0
    lax.fori_loop(0, ZB // 16, _z, 0)

    def _graph(gi, _):
        g = cid * gpc + gi
        pltpu.sync_copy(src_hbm.at[g, sid], src_v)
        pltpu.sync_copy(dst_hbm.at[g, sid], dst_v)
        pltpu.sync_copy(val_hbm.at[g, sid], val_v)
        base = sid * SLICE
        for j in range(SLICE // ZB):
            pltpu.sync_copy(zbuf, acc_shared.at[pl.ds(base + j * ZB, ZB)])
        for j in range(NCH):
            def _idx(k, _, j=j):
                sl = pl.ds(k * 16, 16)
                idx_v[j, sl] = dst_v[j, sl] * NP + src_v[j, sl]
                return 0
            lax.fori_loop(0, CHUNK // 16, _idx, 0)
        plsc.subcore_barrier()
        for j in range(NCH):
            pltpu.sync_copy(val_v.at[j], acc_shared.at[idx_v.at[j]], add=True)
        plsc.subcore_barrier()
        rbase = sid * (NP // NTILE)

        def _fire(r, _):
            pltpu.async_copy(acc_shared.at[pl.ds((rbase + r) * NP, NP)],
                             out_hbm.at[g, rbase + r], sem)
            return 0
        lax.fori_loop(0, NP // NTILE, _fire, 0)

        def _drain(r, _):
            pltpu.make_async_copy(
                acc_shared.at[pl.ds((rbase + r) * NP, NP)],
                out_hbm.at[g, rbase + r], sem).wait()
            return 0
        lax.fori_loop(0, NP // NTILE, _drain, 0)
        return 0

    lax.fori_loop(0, gpc, _graph, 0)


def _sc_build(srcp, dstp, valp):
    nb = srcp.shape[0]
    mesh = plsc.VectorSubcoreMesh(core_axis_name="c", subcore_axis_name="s")
    return pl.kernel(
        _sc_body,
        out_type=jax.ShapeDtypeStruct((nb, NP, NP), jnp.float32),
        mesh=mesh,
        scratch_types=[
            pltpu.VMEM((NCH, CHUNK), jnp.int32),
            pltpu.VMEM((NCH, CHUNK), jnp.int32),
            pltpu.VMEM((NCH, CHUNK), jnp.float32),
            pltpu.VMEM((NCH, CHUNK), jnp.int32),
            pltpu.VMEM((ZB,), jnp.float32),
            pltpu.VMEM_SHARED((NP * NP,), jnp.float32),
            pltpu.SemaphoreType.DMA,
        ],
    )(srcp, dstp, valp)


def _count_ge(key, trial):
    return jnp.sum(jnp.where(key >= trial, jnp.int32(1), jnp.int32(0)))


def _tc_body(nvalid, kth, wm_ref, x_ref, w0_ref, b0_ref, w1_ref, b1_ref,
             w2_ref, b2_ref, pw_ref, m1_ref, mb1_ref, m2_ref, mb2_ref,
             out_ref):
    f32 = jnp.float32
    wm = wm_ref[0]
    ones = jnp.ones((NP, 1), f32)
    deg_row = jnp.sum(wm, axis=0, keepdims=True)
    deg_col = lax.dot_general(wm, ones, (((0,), (0,)), ((), ())))
    dis_row = jnp.where(deg_row > 0,
                        lax.rsqrt(jnp.where(deg_row > 0, deg_row, 1.0)), 0.0)
    dis_col = jnp.where(deg_col > 0,
                        lax.rsqrt(jnp.where(deg_col > 0, deg_col, 1.0)), 0.0)
    a = -(wm * dis_row) * dis_col

    h = x_ref[0]
    for w_ref, b_ref, last in ((w0_ref, b0_ref, False),
                               (w1_ref, b1_ref, False),
                               (w2_ref, b2_ref, True)):
        tx0 = h
        tx1 = jnp.dot(a, tx0, preferred_element_type=f32)
        tx2 = 2.0 * jnp.dot(a, tx1, preferred_element_type=f32) - tx0
        out = (jnp.dot(tx0, w_ref[0], preferred_element_type=f32)
               + jnp.dot(tx1, w_ref[1], preferred_element_type=f32)
               + jnp.dot(tx2, w_ref[2], preferred_element_type=f32)
               + b_ref[...])
        h = out if last else jnp.maximum(out, 0.0)

    pw = pw_ref[...]
    invn = lax.rsqrt(jnp.sum(pw * pw))
    s = jnp.tanh(lax.dot_general(h, pw, (((1,), (1,)), ((), ()))) * invn)
    bits = lax.bitcast_convert_type(s, jnp.int32)
    key = jnp.where(bits < 0, jnp.bitwise_xor(bits, jnp.int32(0x7FFFFFFF)),
                    bits)
    row = lax.broadcasted_iota(jnp.int32, (NP, 1), 0)
    int_min = jnp.int32(-2147483647 - 1)
    key = jnp.where(row < nvalid, key, int_min)

    cpos = _count_ge(key, jnp.int32(0))
    ans0 = jnp.where(cpos >= kth, jnp.int32(0), int_min)

    def _bit(i, ans):
        trial = ans + lax.shift_left(jnp.int32(1), jnp.int32(30) - i)
        return jnp.where(_count_ge(key, trial) >= kth, trial, ans)
    thresh = lax.fori_loop(0, 31, _bit, ans0)

    cgt = _count_ge(key, thresh + 1)
    need = jnp.int32(kth) - cgt
    eq = key == thresh

    def _f(j):
        return jnp.sum(jnp.where(eq & (row <= j), jnp.int32(1), jnp.int32(0)))

    def _jbit(i, j):
        cand = j + lax.shift_left(jnp.int32(1), jnp.int32(10) - i)
        return jnp.where(_f(cand - 1) < need, cand, j)
    jsel = lax.fori_loop(0, 11, _jbit, jnp.int32(0))

    keep = (key > thresh) | (eq & (row <= jsel))
    w = jnp.where(keep, s, 0.0) * (1.0 / kth)
    g_row = lax.dot_general(w, h, (((0,), (0,)), ((), ())))

    z = jnp.maximum(jnp.dot(g_row, m1_ref[...], preferred_element_type=f32)
                    + mb1_ref[...], 0.0)
    o = jnp.dot(z, m2_ref[...], preferred_element_type=f32) + mb2_ref[...]
    out_ref[0] = jnp.broadcast_to(o, (8, 128))


def _tc_call(wm, xp, w0, b0, w1, b1, w2, b2, pw, m1, mb1, m2p, mb2p,
             nvalid, kth):
    nb = wm.shape[0]
    d_in = xp.shape[2]
    full = lambda shp: pl.BlockSpec(shp, lambda g: (0,) * len(shp))
    grid_spec = pl.GridSpec(
        grid=(nb,),
        in_specs=[
            pl.BlockSpec((1, NP, NP), lambda g: (g, 0, 0)),
            pl.BlockSpec((1, NP, d_in), lambda g: (g, 0, 0)),
            full(w0.shape), full(b0.shape), full(w1.shape), full(b1.shape),
            full(w2.shape), full(b2.shape), full(pw.shape), full(m1.shape),
            full(mb1.shape), full(m2p.shape), full(mb2p.shape),
        ],
        out_specs=pl.BlockSpec((1, 8, 128), lambda g: (g, 0, 0)),
    )
    body = functools.partial(_tc_body, nvalid, kth)
    return pl.pallas_call(
        body,
        grid_spec=grid_spec,
        out_shape=jax.ShapeDtypeStruct((nb, 8, 128), jnp.float32),
    )(wm, xp, w0, b0, w1, b1, w2, b2, pw, m1, mb1, m2p, mb2p)


def kernel(freq, edge_index, edge_weight, cheb_W0, cheb_b0, cheb_W1, cheb_b1,
           cheb_W2, cheb_b2, pool_w, mlp_W1, mlp_b1, mlp_W2, mlp_b2):
    nb, n, d = freq.shape
    e = edge_index.shape[2]
    ep = NTILE * NCH * CHUNK
    kth = int(math.ceil(0.5 * n))
    ncls = mlp_W2.shape[1]

    src = edge_index[:, 0, :].astype(jnp.int32)
    dst = edge_index[:, 1, :].astype(jnp.int32)
    srcp = jnp.pad(src, ((0, 0), (0, ep - e))).reshape(nb, NTILE, NCH, CHUNK)
    dstp = jnp.pad(dst, ((0, 0), (0, ep - e))).reshape(nb, NTILE, NCH, CHUNK)
    valp = jnp.pad(edge_weight, ((0, 0), (0, ep - e))).reshape(
        nb, NTILE, NCH, CHUNK)

    xp = jnp.pad(freq, ((0, 0), (0, NP - n), (0, 0)))
    m2p = jnp.pad(mlp_W2, ((0, 0), (0, 128 - ncls)))
    mb2p = jnp.pad(mlp_b2, ((0, 128 - ncls),)).reshape(1, 128)

    half = nb // 2
    outs = []
    wms = [_sc_build(srcp[i * half:(i + 1) * half],
                     dstp[i * half:(i + 1) * half],
                     valp[i * half:(i + 1) * half]) for i in range(2)]
    for i in range(2):
        outs.append(_tc_call(wms[i], xp[i * half:(i + 1) * half],
                             cheb_W0, cheb_b0.reshape(1, -1),
                             cheb_W1, cheb_b1.reshape(1, -1),
                             cheb_W2, cheb_b2.reshape(1, -1),
                             pool_w.reshape(1, -1),
                             mlp_W1, mlp_b1.reshape(1, -1), m2p, mb2p,
                             n, kth))
    out = jnp.concatenate(outs, axis=0)
    return out[:, 0, :ncls]


# speedup vs baseline: 76.4636x; 1.0601x over previous
import functools
import math

import jax
import jax.numpy as jnp
from jax import lax
from jax.experimental import pallas as pl
from jax.experimental.pallas import tpu as pltpu
from jax.experimental.pallas import tpu_sc as plsc

NP = 1280
NTILE = 16
NCORE = 2
CHUNK = 128
NCH = 20
ZB = 12800
SLICE = NP * NP // NTILE


def _sc_body(src_hbm, dst_hbm, val_hbm, out_hbm,
             src_v, dst_v, val_v, idx_v, zbuf, acc_shared, sem):
    cid = lax.axis_index("c")
    sid = lax.axis_index("s")
    nb = src_hbm.shape[0]
    gpc = nb // NCORE

    def _z(i, _):
        zbuf[pl.ds(i * 16, 16)] = jnp.zeros((16,), jnp.float32)
        return 0
    lax.fori_loop(0, ZB // 16, _z, 0)

    def _graph(gi, _):
        g = cid * gpc + gi
        pltpu.sync_copy(src_hbm.at[g, sid], src_v)
        pltpu.sync_copy(dst_hbm.at[g, sid], dst_v)
        pltpu.sync_copy(val_hbm.at[g, sid], val_v)
        base = sid * SLICE
        for j in range(SLICE // ZB):
            pltpu.sync_copy(zbuf, acc_shared.at[pl.ds(base + j * ZB, ZB)])
        for j in range(NCH):
            def _idx(k, _, j=j):
                sl = pl.ds(k * 16, 16)
                idx_v[j, sl] = dst_v[j, sl] * NP + src_v[j, sl]
                return 0
            lax.fori_loop(0, CHUNK // 16, _idx, 0)
        plsc.subcore_barrier()
        for j in range(NCH):
            pltpu.sync_copy(val_v.at[j], acc_shared.at[idx_v.at[j]], add=True)
        plsc.subcore_barrier()
        rbase = sid * (NP // NTILE)

        def _fire(r, _):
            pltpu.async_copy(acc_shared.at[pl.ds((rbase + r) * NP, NP)],
                             out_hbm.at[g, rbase + r], sem)
            return 0
        lax.fori_loop(0, NP // NTILE, _fire, 0)

        def _drain(r, _):
            pltpu.make_async_copy(
                acc_shared.at[pl.ds((rbase + r) * NP, NP)],
                out_hbm.at[g, rbase + r], sem).wait()
            return 0
        lax.fori_loop(0, NP // NTILE, _drain, 0)
        return 0

    lax.fori_loop(0, gpc, _graph, 0)


def _sc_build(srcp, dstp, valp):
    nb = srcp.shape[0]
    mesh = plsc.VectorSubcoreMesh(core_axis_name="c", subcore_axis_name="s")
    return pl.kernel(
        _sc_body,
        out_type=jax.ShapeDtypeStruct((nb, NP, NP), jnp.float32),
        mesh=mesh,
        scratch_types=[
            pltpu.VMEM((NCH, CHUNK), jnp.int32),
            pltpu.VMEM((NCH, CHUNK), jnp.int32),
            pltpu.VMEM((NCH, CHUNK), jnp.float32),
            pltpu.VMEM((NCH, CHUNK), jnp.int32),
            pltpu.VMEM((ZB,), jnp.float32),
            pltpu.VMEM_SHARED((NP * NP,), jnp.float32),
            pltpu.SemaphoreType.DMA,
        ],
    )(srcp, dstp, valp)


def _count_ge(key, trial):
    return jnp.sum(jnp.where(key >= trial, jnp.int32(1), jnp.int32(0)))


def _tc_body(nvalid, kth, wm_ref, x_ref, w0_ref, b0_ref, w1_ref, b1_ref,
             w2_ref, b2_ref, pw_ref, m1_ref, mb1_ref, m2_ref, mb2_ref,
             out_ref):
    f32 = jnp.float32
    wm = wm_ref[...]
    deg_row = jnp.sum(wm, axis=0, keepdims=True)
    dis_row = jnp.where(deg_row > 0,
                        lax.rsqrt(jnp.where(deg_row > 0, deg_row, 1.0)), 0.0)
    dis_col = jnp.transpose(dis_row)
    a = -(wm * dis_row) * dis_col

    h = x_ref[...]
    for w_ref, b_ref, last in ((w0_ref, b0_ref, False),
                               (w1_ref, b1_ref, False),
                               (w2_ref, b2_ref, True)):
        tx0 = h
        tx1 = jnp.dot(a, tx0, preferred_element_type=f32)
        tx2 = 2.0 * jnp.dot(a, tx1, preferred_element_type=f32) - tx0
        out = (jnp.dot(tx0, w_ref[0], preferred_element_type=f32)
               + jnp.dot(tx1, w_ref[1], preferred_element_type=f32)
               + jnp.dot(tx2, w_ref[2], preferred_element_type=f32)
               + b_ref[...])
        h = out if last else jnp.maximum(out, 0.0)

    pw = pw_ref[...]
    invn = lax.rsqrt(jnp.sum(pw * pw))
    s = jnp.tanh(lax.dot_general(h, pw, (((1,), (1,)), ((), ()))) * invn)
    bits = lax.bitcast_convert_type(s, jnp.int32)
    key = jnp.where(bits < 0, jnp.bitwise_xor(bits, jnp.int32(0x7FFFFFFF)),
                    bits)
    row = lax.broadcasted_iota(jnp.int32, (NP, 1), 0)
    int_min = jnp.int32(-2147483647 - 1)
    key = jnp.where(row < nvalid, key, int_min)

    cpos = _count_ge(key, jnp.int32(0))
    ans0 = jnp.where(cpos >= kth, jnp.int32(0), int_min)

    def _bit(i, ans):
        trial = ans + lax.shift_left(jnp.int32(1), jnp.int32(30) - i)
        return jnp.where(_count_ge(key, trial) >= kth, trial, ans)
    thresh = lax.fori_loop(0, 31, _bit, ans0)

    cgt = _count_ge(key, thresh + 1)
    need = jnp.int32(kth) - cgt
    eq = key == thresh

    def _f(j):
        return jnp.sum(jnp.where(eq & (row <= j), jnp.int32(1), jnp.int32(0)))

    def _jbit(i, j):
        cand = j + lax.shift_left(jnp.int32(1), jnp.int32(10) - i)
        return jnp.where(_f(cand - 1) < need, cand, j)
    jsel = lax.fori_loop(0, 11, _jbit, jnp.int32(0))

    keep = (key > thresh) | (eq & (row <= jsel))
    w = jnp.where(keep, s, 0.0) * (1.0 / kth)
    g_row = lax.dot_general(w, h, (((0,), (0,)), ((), ())))

    z = jnp.maximum(jnp.dot(g_row, m1_ref[...], preferred_element_type=f32)
                    + mb1_ref[...], 0.0)
    o = jnp.dot(z, m2_ref[...], preferred_element_type=f32) + mb2_ref[...]
    out_ref[...] = jnp.broadcast_to(o, (8, 128))


def _tc_call(wm, xp, w0, b0, w1, b1, w2, b2, pw, m1, mb1, m2p, mb2p,
             nvalid, kth):
    nb = wm.shape[0]
    d_in = xp.shape[2]
    full = lambda shp: pl.BlockSpec(shp, lambda g: (0,) * len(shp))
    grid_spec = pl.GridSpec(
        grid=(nb,),
        in_specs=[
            pl.BlockSpec((None, NP, NP), lambda g: (g, 0, 0)),
            pl.BlockSpec((None, NP, d_in), lambda g: (g, 0, 0)),
            full(w0.shape), full(b0.shape), full(w1.shape), full(b1.shape),
            full(w2.shape), full(b2.shape), full(pw.shape), full(m1.shape),
            full(mb1.shape), full(m2p.shape), full(mb2p.shape),
        ],
        out_specs=pl.BlockSpec((None, 8, 128), lambda g: (g, 0, 0)),
    )
    body = functools.partial(_tc_body, nvalid, kth)
    return pl.pallas_call(
        body,
        grid_spec=grid_spec,
        out_shape=jax.ShapeDtypeStruct((nb, 8, 128), jnp.float32),
    )(wm, xp, w0, b0, w1, b1, w2, b2, pw, m1, mb1, m2p, mb2p)


def kernel(freq, edge_index, edge_weight, cheb_W0, cheb_b0, cheb_W1, cheb_b1,
           cheb_W2, cheb_b2, pool_w, mlp_W1, mlp_b1, mlp_W2, mlp_b2):
    nb, n, d = freq.shape
    e = edge_index.shape[2]
    ep = NTILE * NCH * CHUNK
    kth = int(math.ceil(0.5 * n))
    ncls = mlp_W2.shape[1]

    src = edge_index[:, 0, :].astype(jnp.int32)
    dst = edge_index[:, 1, :].astype(jnp.int32)
    srcp = jnp.pad(src, ((0, 0), (0, ep - e))).reshape(nb, NTILE, NCH, CHUNK)
    dstp = jnp.pad(dst, ((0, 0), (0, ep - e))).reshape(nb, NTILE, NCH, CHUNK)
    valp = jnp.pad(edge_weight, ((0, 0), (0, ep - e))).reshape(
        nb, NTILE, NCH, CHUNK)

    xp = jnp.pad(freq, ((0, 0), (0, NP - n), (0, 0)))
    m2p = jnp.pad(mlp_W2, ((0, 0), (0, 128 - ncls)))
    mb2p = jnp.pad(mlp_b2, ((0, 128 - ncls),)).reshape(1, 128)

    half = nb // 2
    outs = []
    wms = [_sc_build(srcp[i * half:(i + 1) * half],
                     dstp[i * half:(i + 1) * half],
                     valp[i * half:(i + 1) * half]) for i in range(2)]
    for i in range(2):
        outs.append(_tc_call(wms[i], xp[i * half:(i + 1) * half],
                             cheb_W0, cheb_b0.reshape(1, -1),
                             cheb_W1, cheb_b1.reshape(1, -1),
                             cheb_W2, cheb_b2.reshape(1, -1),
                             pool_w.reshape(1, -1),
                             mlp_W1, mlp_b1.reshape(1, -1), m2p, mb2p,
                             n, kth))
    out = jnp.concatenate(outs, axis=0)
    return out[:, 0, :ncls]


# speedup vs baseline: 77.7223x; 1.0165x over previous
import functools
import math

import jax
import jax.numpy as jnp
from jax import lax
from jax.experimental import pallas as pl
from jax.experimental.pallas import tpu as pltpu
from jax.experimental.pallas import tpu_sc as plsc

NP = 1280
NTILE = 16
NCORE = 2
CHUNK = 128
NCH = 20
ZB = 12800
SLICE = NP * NP // NTILE


def _sc_body(src_hbm, dst_hbm, val_hbm, out_hbm,
             src_v, dst_v, val_v, idx_v, zbuf, acc_shared, sem):
    cid = lax.axis_index("c")
    sid = lax.axis_index("s")
    nb = src_hbm.shape[0]
    gpc = nb // NCORE

    def _z(i, _):
        zbuf[pl.ds(i * 16, 16)] = jnp.zeros((16,), jnp.float32)
        return 0
    lax.fori_loop(0, ZB // 16, _z, 0)

    def _graph(gi, _):
        g = cid * gpc + gi
        pltpu.sync_copy(src_hbm.at[g, sid], src_v)
        pltpu.sync_copy(dst_hbm.at[g, sid], dst_v)
        pltpu.sync_copy(val_hbm.at[g, sid], val_v)
        base = sid * SLICE
        for j in range(SLICE // ZB):
            pltpu.sync_copy(zbuf, acc_shared.at[pl.ds(base + j * ZB, ZB)])
        for j in range(NCH):
            def _idx(k, _, j=j):
                sl = pl.ds(k * 16, 16)
                idx_v[j, sl] = dst_v[j, sl] * NP + src_v[j, sl]
                return 0
            lax.fori_loop(0, CHUNK // 16, _idx, 0)
        plsc.subcore_barrier()
        for j in range(NCH):
            pltpu.sync_copy(val_v.at[j], acc_shared.at[idx_v.at[j]], add=True)
        plsc.subcore_barrier()
        rbase = sid * (NP // NTILE)

        def _fire(r, _):
            pltpu.async_copy(acc_shared.at[pl.ds((rbase + r) * NP, NP)],
                             out_hbm.at[g, rbase + r], sem)
            return 0
        lax.fori_loop(0, NP // NTILE, _fire, 0)

        def _drain(r, _):
            pltpu.make_async_copy(
                acc_shared.at[pl.ds((rbase + r) * NP, NP)],
                out_hbm.at[g, rbase + r], sem).wait()
            return 0
        lax.fori_loop(0, NP // NTILE, _drain, 0)
        return 0

    lax.fori_loop(0, gpc, _graph, 0)


def _sc_build(srcp, dstp, valp):
    nb = srcp.shape[0]
    mesh = plsc.VectorSubcoreMesh(core_axis_name="c", subcore_axis_name="s")
    return pl.kernel(
        _sc_body,
        out_type=jax.ShapeDtypeStruct((nb, NP, NP), jnp.float32),
        mesh=mesh,
        scratch_types=[
            pltpu.VMEM((NCH, CHUNK), jnp.int32),
            pltpu.VMEM((NCH, CHUNK), jnp.int32),
            pltpu.VMEM((NCH, CHUNK), jnp.float32),
            pltpu.VMEM((NCH, CHUNK), jnp.int32),
            pltpu.VMEM((ZB,), jnp.float32),
            pltpu.VMEM_SHARED((NP * NP,), jnp.float32),
            pltpu.SemaphoreType.DMA,
        ],
    )(srcp, dstp, valp)


def _count_ge(key, trial):
    return jnp.sum(jnp.where(key >= trial, jnp.int32(1), jnp.int32(0)))


def _tc_body(nvalid, kth, wm_ref, x_ref, w0_ref, b0_ref, w1_ref, b1_ref,
             w2_ref, b2_ref, pw_ref, m1_ref, mb1_ref, m2_ref, mb2_ref,
             out_ref):
    f32 = jnp.float32
    wm = wm_ref[...]
    deg_row = jnp.sum(wm, axis=0, keepdims=True)
    dis_row = jnp.where(deg_row > 0,
                        lax.rsqrt(jnp.where(deg_row > 0, deg_row, 1.0)), 0.0)
    dis_col = jnp.transpose(dis_row)
    a = -(wm * dis_row) * dis_col

    h = x_ref[...]
    for w_ref, b_ref, last in ((w0_ref, b0_ref, False),
                               (w1_ref, b1_ref, False),
                               (w2_ref, b2_ref, True)):
        tx0 = h
        tx1 = jnp.dot(a, tx0, preferred_element_type=f32)
        tx2 = 2.0 * jnp.dot(a, tx1, preferred_element_type=f32) - tx0
        out = (jnp.dot(tx0, w_ref[0], preferred_element_type=f32)
               + jnp.dot(tx1, w_ref[1], preferred_element_type=f32)
               + jnp.dot(tx2, w_ref[2], preferred_element_type=f32)
               + b_ref[...])
        h = out if last else jnp.maximum(out, 0.0)

    pw = pw_ref[...]
    invn = lax.rsqrt(jnp.sum(pw * pw))
    s = jnp.tanh(lax.dot_general(h, pw, (((1,), (1,)), ((), ()))) * invn)
    bits = lax.bitcast_convert_type(s, jnp.int32)
    key = jnp.where(bits < 0, jnp.bitwise_xor(bits, jnp.int32(0x7FFFFFFF)),
                    bits)
    row = lax.broadcasted_iota(jnp.int32, (NP, 1), 0)
    int_min = jnp.int32(-2147483647 - 1)
    key = jnp.where(row < nvalid, key, int_min)

    cpos = _count_ge(key, jnp.int32(0))
    ans0 = jnp.where(cpos >= kth, jnp.int32(0), int_min)

    def _bit(i, ans):
        trial = ans + lax.shift_left(jnp.int32(1), jnp.int32(30) - i)
        return jnp.where(_count_ge(key, trial) >= kth, trial, ans)
    thresh = lax.fori_loop(0, 31, _bit, ans0)

    cgt = _count_ge(key, thresh + 1)
    need = jnp.int32(kth) - cgt
    eq = key == thresh

    def _f(j):
        return jnp.sum(jnp.where(eq & (row <= j), jnp.int32(1), jnp.int32(0)))

    def _jbit(i, j):
        cand = j + lax.shift_left(jnp.int32(1), jnp.int32(10) - i)
        return jnp.where(_f(cand - 1) < need, cand, j)
    jsel = lax.fori_loop(0, 11, _jbit, jnp.int32(0))

    keep = (key > thresh) | (eq & (row <= jsel))
    w = jnp.where(keep, s, 0.0) * (1.0 / kth)
    g_row = lax.dot_general(w, h, (((0,), (0,)), ((), ())))

    z = jnp.maximum(jnp.dot(g_row, m1_ref[...], preferred_element_type=f32)
                    + mb1_ref[...], 0.0)
    o = jnp.dot(z, m2_ref[...], preferred_element_type=f32) + mb2_ref[...]
    out_ref[...] = jnp.broadcast_to(o, (8, 128))


def _tc_call(wm, xp, w0, b0, w1, b1, w2, b2, pw, m1, mb1, m2p, mb2p,
             nvalid, kth):
    nb = wm.shape[0]
    d_in = xp.shape[2]
    full = lambda shp: pl.BlockSpec(shp, lambda g: (0,) * len(shp))
    grid_spec = pl.GridSpec(
        grid=(nb,),
        in_specs=[
            pl.BlockSpec((None, NP, NP), lambda g: (g, 0, 0)),
            pl.BlockSpec((None, NP, d_in), lambda g: (g, 0, 0)),
            full(w0.shape), full(b0.shape), full(w1.shape), full(b1.shape),
            full(w2.shape), full(b2.shape), full(pw.shape), full(m1.shape),
            full(mb1.shape), full(m2p.shape), full(mb2p.shape),
        ],
        out_specs=pl.BlockSpec((None, 8, 128), lambda g: (g, 0, 0)),
    )
    body = functools.partial(_tc_body, nvalid, kth)
    return pl.pallas_call(
        body,
        grid_spec=grid_spec,
        out_shape=jax.ShapeDtypeStruct((nb, 8, 128), jnp.float32),
    )(wm, xp, w0, b0, w1, b1, w2, b2, pw, m1, mb1, m2p, mb2p)


def kernel(freq, edge_index, edge_weight, cheb_W0, cheb_b0, cheb_W1, cheb_b1,
           cheb_W2, cheb_b2, pool_w, mlp_W1, mlp_b1, mlp_W2, mlp_b2):
    nb, n, d = freq.shape
    e = edge_index.shape[2]
    ep = NTILE * NCH * CHUNK
    kth = int(math.ceil(0.5 * n))
    ncls = mlp_W2.shape[1]

    src = edge_index[:, 0, :].astype(jnp.int32)
    dst = edge_index[:, 1, :].astype(jnp.int32)
    srcp = jnp.pad(src, ((0, 0), (0, ep - e))).reshape(nb, NTILE, NCH, CHUNK)
    dstp = jnp.pad(dst, ((0, 0), (0, ep - e))).reshape(nb, NTILE, NCH, CHUNK)
    valp = jnp.pad(edge_weight, ((0, 0), (0, ep - e))).reshape(
        nb, NTILE, NCH, CHUNK)

    xp = jnp.pad(freq, ((0, 0), (0, NP - n), (0, 0)))
    m2p = jnp.pad(mlp_W2, ((0, 0), (0, 128 - ncls)))
    mb2p = jnp.pad(mlp_b2, ((0, 128 - ncls),)).reshape(1, 128)

    nchunk = 4
    half = nb // nchunk
    outs = []
    wms = [_sc_build(srcp[i * half:(i + 1) * half],
                     dstp[i * half:(i + 1) * half],
                     valp[i * half:(i + 1) * half]) for i in range(nchunk)]
    for i in range(nchunk):
        outs.append(_tc_call(wms[i], xp[i * half:(i + 1) * half],
                             cheb_W0, cheb_b0.reshape(1, -1),
                             cheb_W1, cheb_b1.reshape(1, -1),
                             cheb_W2, cheb_b2.reshape(1, -1),
                             pool_w.reshape(1, -1),
                             mlp_W1, mlp_b1.reshape(1, -1), m2p, mb2p,
                             n, kth))
    out = jnp.concatenate(outs, axis=0)
    return out[:, 0, :ncls]


# speedup vs baseline: 81.7657x; 1.0520x over previous
import functools
import math

import jax
import jax.numpy as jnp
from jax import lax
from jax.experimental import pallas as pl
from jax.experimental.pallas import tpu as pltpu
from jax.experimental.pallas import tpu_sc as plsc

NP = 1280
NTILE = 16
NCORE = 2
CHUNK = 128
NCH = 20
ZB = 12800
SLICE = NP * NP // NTILE


def _sc_body(src_hbm, dst_hbm, val_hbm, out_hbm,
             src_v, dst_v, val_v, idx_v, zbuf, acc_shared, sem):
    cid = lax.axis_index("c")
    sid = lax.axis_index("s")
    nb = src_hbm.shape[0]
    gpc = nb // NCORE

    def _z(i, _):
        zbuf[pl.ds(i * 16, 16)] = jnp.zeros((16,), jnp.float32)
        return 0
    lax.fori_loop(0, ZB // 16, _z, 0)

    def _graph(gi, _):
        g = cid * gpc + gi
        pltpu.sync_copy(src_hbm.at[g, sid], src_v)
        pltpu.sync_copy(dst_hbm.at[g, sid], dst_v)
        pltpu.sync_copy(val_hbm.at[g, sid], val_v)
        base = sid * SLICE
        for j in range(SLICE // ZB):
            pltpu.sync_copy(zbuf, acc_shared.at[pl.ds(base + j * ZB, ZB)])
        for j in range(NCH):
            def _idx(k, _, j=j):
                sl = pl.ds(k * 16, 16)
                idx_v[j, sl] = dst_v[j, sl] * NP + src_v[j, sl]
                return 0
            lax.fori_loop(0, CHUNK // 16, _idx, 0)
        plsc.subcore_barrier()
        for j in range(NCH):
            pltpu.sync_copy(val_v.at[j], acc_shared.at[idx_v.at[j]], add=True)
        plsc.subcore_barrier()
        rbase = sid * (NP // NTILE)

        def _fire(r, _):
            pltpu.async_copy(acc_shared.at[pl.ds((rbase + r) * NP, NP)],
                             out_hbm.at[g, rbase + r], sem)
            return 0
        lax.fori_loop(0, NP // NTILE, _fire, 0)

        def _drain(r, _):
            pltpu.make_async_copy(
                acc_shared.at[pl.ds((rbase + r) * NP, NP)],
                out_hbm.at[g, rbase + r], sem).wait()
            return 0
        lax.fori_loop(0, NP // NTILE, _drain, 0)
        return 0

    lax.fori_loop(0, gpc, _graph, 0)


def _sc_build(srcp, dstp, valp):
    nb = srcp.shape[0]
    mesh = plsc.VectorSubcoreMesh(core_axis_name="c", subcore_axis_name="s")
    return pl.kernel(
        _sc_body,
        out_type=jax.ShapeDtypeStruct((nb, NP, NP), jnp.float32),
        mesh=mesh,
        scratch_types=[
            pltpu.VMEM((NCH, CHUNK), jnp.int32),
            pltpu.VMEM((NCH, CHUNK), jnp.int32),
            pltpu.VMEM((NCH, CHUNK), jnp.float32),
            pltpu.VMEM((NCH, CHUNK), jnp.int32),
            pltpu.VMEM((ZB,), jnp.float32),
            pltpu.VMEM_SHARED((NP * NP,), jnp.float32),
            pltpu.SemaphoreType.DMA,
        ],
    )(srcp, dstp, valp)


def _count_ge(key, trial):
    return jnp.sum(jnp.where(key >= trial, jnp.int32(1), jnp.int32(0)))


def _tc_body(nvalid, kth, wm_ref, x_ref, w0_ref, b0_ref, w1_ref, b1_ref,
             w2_ref, b2_ref, pw_ref, m1_ref, mb1_ref, m2_ref, mb2_ref,
             out_ref):
    f32 = jnp.float32
    wm = wm_ref[...]
    deg_row = jnp.sum(wm, axis=0, keepdims=True)
    dis_row = jnp.where(deg_row > 0,
                        lax.rsqrt(jnp.where(deg_row > 0, deg_row, 1.0)), 0.0)
    dis_col = jnp.transpose(dis_row)
    a = -(wm * dis_row) * dis_col

    h = x_ref[...]
    for w_ref, b_ref, last in ((w0_ref, b0_ref, False),
                               (w1_ref, b1_ref, False),
                               (w2_ref, b2_ref, True)):
        tx0 = h
        tx1 = jnp.dot(a, tx0, preferred_element_type=f32)
        tx2 = 2.0 * jnp.dot(a, tx1, preferred_element_type=f32) - tx0
        out = (jnp.dot(tx0, w_ref[0], preferred_element_type=f32)
               + jnp.dot(tx1, w_ref[1], preferred_element_type=f32)
               + jnp.dot(tx2, w_ref[2], preferred_element_type=f32)
               + b_ref[...])
        h = out if last else jnp.maximum(out, 0.0)

    pw = pw_ref[...]
    invn = lax.rsqrt(jnp.sum(pw * pw))
    s = jnp.tanh(lax.dot_general(pw, h, (((1,), (1,)), ((), ()))) * invn)
    bits = lax.bitcast_convert_type(s, jnp.int32)
    key = jnp.where(bits < 0, jnp.bitwise_xor(bits, jnp.int32(0x7FFFFFFF)),
                    bits)
    row = lax.broadcasted_iota(jnp.int32, (1, NP), 1)
    int_min = jnp.int32(-2147483647 - 1)
    key = jnp.where(row < nvalid, key, int_min)

    cpos = _count_ge(key, jnp.int32(0))
    ans0 = jnp.where(cpos >= kth, jnp.int32(0), int_min)

    def _bit(i, ans):
        trial = ans + lax.shift_left(jnp.int32(1), jnp.int32(30) - i)
        return jnp.where(_count_ge(key, trial) >= kth, trial, ans)
    thresh = lax.fori_loop(0, 31, _bit, ans0)

    cgt = _count_ge(key, thresh + 1)
    need = jnp.int32(kth) - cgt
    eq = key == thresh

    def _f(j):
        return jnp.sum(jnp.where(eq & (row <= j), jnp.int32(1), jnp.int32(0)))

    def _jbit(i, j):
        cand = j + lax.shift_left(jnp.int32(1), jnp.int32(10) - i)
        return jnp.where(_f(cand - 1) < need, cand, j)
    jsel = lax.fori_loop(0, 11, _jbit, jnp.int32(0))

    keep = (key > thresh) | (eq & (row <= jsel))
    w = jnp.where(keep, s, 0.0) * (1.0 / kth)
    g_row = lax.dot_general(w, h, (((1,), (0,)), ((), ())))

    z = jnp.maximum(jnp.dot(g_row, m1_ref[...], preferred_element_type=f32)
                    + mb1_ref[...], 0.0)
    o = jnp.dot(z, m2_ref[...], preferred_element_type=f32) + mb2_ref[...]
    out_ref[...] = jnp.broadcast_to(o, (8, 128))


def _tc_call(wm, xp, w0, b0, w1, b1, w2, b2, pw, m1, mb1, m2p, mb2p,
             nvalid, kth):
    nb = wm.shape[0]
    d_in = xp.shape[2]
    full = lambda shp: pl.BlockSpec(shp, lambda g: (0,) * len(shp))
    grid_spec = pl.GridSpec(
        grid=(nb,),
        in_specs=[
            pl.BlockSpec((None, NP, NP), lambda g: (g, 0, 0)),
            pl.BlockSpec((None, NP, d_in), lambda g: (g, 0, 0)),
            full(w0.shape), full(b0.shape), full(w1.shape), full(b1.shape),
            full(w2.shape), full(b2.shape), full(pw.shape), full(m1.shape),
            full(mb1.shape), full(m2p.shape), full(mb2p.shape),
        ],
        out_specs=pl.BlockSpec((None, 8, 128), lambda g: (g, 0, 0)),
    )
    body = functools.partial(_tc_body, nvalid, kth)
    return pl.pallas_call(
        body,
        grid_spec=grid_spec,
        out_shape=jax.ShapeDtypeStruct((nb, 8, 128), jnp.float32),
    )(wm, xp, w0, b0, w1, b1, w2, b2, pw, m1, mb1, m2p, mb2p)


def kernel(freq, edge_index, edge_weight, cheb_W0, cheb_b0, cheb_W1, cheb_b1,
           cheb_W2, cheb_b2, pool_w, mlp_W1, mlp_b1, mlp_W2, mlp_b2):
    nb, n, d = freq.shape
    e = edge_index.shape[2]
    ep = NTILE * NCH * CHUNK
    kth = int(math.ceil(0.5 * n))
    ncls = mlp_W2.shape[1]

    src = edge_index[:, 0, :].astype(jnp.int32)
    dst = edge_index[:, 1, :].astype(jnp.int32)
    srcp = jnp.pad(src, ((0, 0), (0, ep - e))).reshape(nb, NTILE, NCH, CHUNK)
    dstp = jnp.pad(dst, ((0, 0), (0, ep - e))).reshape(nb, NTILE, NCH, CHUNK)
    valp = jnp.pad(edge_weight, ((0, 0), (0, ep - e))).reshape(
        nb, NTILE, NCH, CHUNK)

    xp = jnp.pad(freq, ((0, 0), (0, NP - n), (0, 0)))
    m2p = jnp.pad(mlp_W2, ((0, 0), (0, 128 - ncls)))
    mb2p = jnp.pad(mlp_b2, ((0, 128 - ncls),)).reshape(1, 128)

    nchunk = 4
    half = nb // nchunk
    outs = []
    wms = [_sc_build(srcp[i * half:(i + 1) * half],
                     dstp[i * half:(i + 1) * half],
                     valp[i * half:(i + 1) * half]) for i in range(nchunk)]
    for i in range(nchunk):
        outs.append(_tc_call(wms[i], xp[i * half:(i + 1) * half],
                             cheb_W0, cheb_b0.reshape(1, -1),
                             cheb_W1, cheb_b1.reshape(1, -1),
                             cheb_W2, cheb_b2.reshape(1, -1),
                             pool_w.reshape(1, -1),
                             mlp_W1, mlp_b1.reshape(1, -1), m2p, mb2p,
                             n, kth))
    out = jnp.concatenate(outs, axis=0)
    return out[:, 0, :ncls]
